# baseline (speedup 1.0000x reference)
import jax
import jax.numpy as jnp
from jax.experimental import pallas as pl
from jax.experimental.pallas import tpu as pltpu

_N = 4096
_F = 256
_BM = 512
_NBLK = _N // _BM


def _cheb_kernel(L_ref, xb_ref, w_ref, b_ref, out_ref, Lb_ref, t1_ref, a_ref):
    ph = pl.program_id(0)
    m = pl.program_id(1)

    @pl.when((ph == 0) & (m == 0))
    def _init():
        t1_ref[...] = jnp.zeros((_N, _F), jnp.bfloat16)

    @pl.when(ph == 0)
    def _phase0():
        row = pl.ds(m * _BM, _BM)
        strip = L_ref[...].astype(jnp.bfloat16)
        a_ref[row, :] = jnp.dot(strip, t1_ref[...],
                                preferred_element_type=jnp.float32)
        t1m = jnp.dot(strip, xb_ref[...], preferred_element_type=jnp.float32)
        t1_ref[row, :] = t1m.astype(jnp.bfloat16)
        Lb_ref[row, :] = strip
        t1m_b = t1m.astype(jnp.bfloat16)

        def col_body(i, _):
            r_i = pl.ds(i * _BM, _BM)
            blk = Lb_ref[r_i, pl.ds(m * _BM, _BM)]
            a_ref[r_i, :] += jnp.dot(blk, t1m_b,
                                     preferred_element_type=jnp.float32)
            return 0

        jax.lax.fori_loop(0, m + 1, col_body, 0)

    @pl.when(ph == 1)
    def _phase1():
        row = pl.ds(m * _BM, _BM)
        w0m2 = (w_ref[0, :, :] - w_ref[2, :, :]).astype(jnp.bfloat16)
        w1 = w_ref[1, :, :].astype(jnp.bfloat16)
        w2 = w_ref[2, :, :].astype(jnp.bfloat16)
        acc = jnp.dot(xb_ref[row, :], w0m2, preferred_element_type=jnp.float32)
        acc += jnp.dot(t1_ref[row, :], w1, preferred_element_type=jnp.float32)
        acc += 2.0 * jnp.dot(a_ref[row, :].astype(jnp.bfloat16), w2,
                             preferred_element_type=jnp.float32)
        out_ref[...] = acc + b_ref[...]


def kernel(x, L_tilde, weight, bias):
    xb = x.astype(jnp.bfloat16)
    bias2d = bias.reshape(1, _F)

    out = pl.pallas_call(
        _cheb_kernel,
        grid=(2, _NBLK),
        in_specs=[
            pl.BlockSpec(
                (_BM, _N),
                lambda p, i: (i * (1 - p) + (_NBLK - 1) * p, 0)),
            pl.BlockSpec((_N, _F), lambda p, i: (0, 0)),
            pl.BlockSpec((3, _F, _F), lambda p, i: (0, 0, 0)),
            pl.BlockSpec((1, _F), lambda p, i: (0, 0)),
        ],
        out_specs=pl.BlockSpec((_BM, _F), lambda p, i: (p * i, 0)),
        out_shape=jax.ShapeDtypeStruct((_N, _F), jnp.float32),
        scratch_shapes=[
            pltpu.VMEM((_N, _N), jnp.bfloat16),
            pltpu.VMEM((_N, _F), jnp.bfloat16),
            pltpu.VMEM((_N, _F), jnp.float32),
        ],
        compiler_params=pltpu.CompilerParams(
            dimension_semantics=("arbitrary", "arbitrary"),
        ),
    )(L_tilde, xb, weight, bias2d)
    return out


# speedup vs baseline: 1.1123x; 1.1123x over previous
import jax
import jax.numpy as jnp
from jax.experimental import pallas as pl
from jax.experimental.pallas import tpu as pltpu

_N = 4096
_F = 256
_BM = 512
_BO = 1024
_NBLK = _N // _BM


def _cheb_kernel(L_ref, xb_ref, w_ref, b_ref, out_ref, Lb_ref, t1_ref):
    ph = pl.program_id(0)
    i = pl.program_id(1)

    @pl.when(ph == 0)
    def _phase0():
        row = pl.ds(i * _BM, _BM)
        strip = L_ref[...].astype(jnp.bfloat16)
        Lb_ref[row, :] = strip
        t1 = jnp.dot(strip, xb_ref[...], preferred_element_type=jnp.float32)
        t1_ref[row, :] = t1.astype(jnp.bfloat16)

    @pl.when((ph == 1) & (i % 2 == 0))
    def _phase1():
        row = pl.ds((i // 2) * _BO, _BO)
        w0m2 = (w_ref[0, :, :] - w_ref[2, :, :]).astype(jnp.bfloat16)
        w1 = w_ref[1, :, :].astype(jnp.bfloat16)
        w2 = w_ref[2, :, :].astype(jnp.bfloat16)
        a = jnp.dot(Lb_ref[row, :], t1_ref[...],
                    preferred_element_type=jnp.float32)
        acc = jnp.dot(xb_ref[row, :], w0m2, preferred_element_type=jnp.float32)
        acc += jnp.dot(t1_ref[row, :], w1, preferred_element_type=jnp.float32)
        acc += 2.0 * jnp.dot(a.astype(jnp.bfloat16), w2,
                             preferred_element_type=jnp.float32)
        out_ref[...] = acc + b_ref[...]


def kernel(x, L_tilde, weight, bias):
    xb = x.astype(jnp.bfloat16)
    bias2d = bias.reshape(1, _F)

    out = pl.pallas_call(
        _cheb_kernel,
        grid=(2, _NBLK),
        in_specs=[
            pl.BlockSpec(
                (_BM, _N),
                lambda p, i: (i * (1 - p) + (_NBLK - 1) * p, 0)),
            pl.BlockSpec((_N, _F), lambda p, i: (0, 0)),
            pl.BlockSpec((3, _F, _F), lambda p, i: (0, 0, 0)),
            pl.BlockSpec((1, _F), lambda p, i: (0, 0)),
        ],
        out_specs=pl.BlockSpec((_BO, _F), lambda p, i: (p * (i // 2), 0)),
        out_shape=jax.ShapeDtypeStruct((_N, _F), jnp.float32),
        scratch_shapes=[
            pltpu.VMEM((_N, _N), jnp.bfloat16),
            pltpu.VMEM((_N, _F), jnp.bfloat16),
        ],
        compiler_params=pltpu.CompilerParams(
            dimension_semantics=("arbitrary", "arbitrary"),
        ),
    )(L_tilde, xb, weight, bias2d)
    return out


# speedup vs baseline: 1.1244x; 1.0109x over previous
import jax
import jax.numpy as jnp
from jax.experimental import pallas as pl
from jax.experimental.pallas import tpu as pltpu

_N = 4096
_F = 256
_BM = 512
_BO = 1024
_NBLK = _N // _BM
_NOUT = _N // _BO


def _cheb_kernel(L_ref, xb_ref, w_ref, b_ref, out_ref, Lb_ref, t1_ref):
    m = pl.program_id(0)

    @pl.when(m < _NBLK)
    def _stream():
        row = pl.ds(m * _BM, _BM)
        strip = L_ref[...].astype(jnp.bfloat16)
        Lb_ref[row, :] = strip
        t1 = jnp.dot(strip, xb_ref[...], preferred_element_type=jnp.float32)
        t1_ref[row, :] = t1.astype(jnp.bfloat16)

    @pl.when(m >= _NBLK)
    def _produce():
        row = pl.ds((m - _NBLK) * _BO, _BO)
        w0m2 = (w_ref[0, :, :] - w_ref[2, :, :]).astype(jnp.bfloat16)
        w1 = w_ref[1, :, :].astype(jnp.bfloat16)
        w2 = w_ref[2, :, :].astype(jnp.bfloat16)
        a = jnp.dot(Lb_ref[row, :], t1_ref[...],
                    preferred_element_type=jnp.float32)
        acc = jnp.dot(xb_ref[row, :], w0m2, preferred_element_type=jnp.float32)
        acc += jnp.dot(t1_ref[row, :], w1, preferred_element_type=jnp.float32)
        acc += 2.0 * jnp.dot(a.astype(jnp.bfloat16), w2,
                             preferred_element_type=jnp.float32)
        out_ref[...] = acc + b_ref[...]


def kernel(x, L_tilde, weight, bias):
    xb = x.astype(jnp.bfloat16)
    bias2d = bias.reshape(1, _F)

    out = pl.pallas_call(
        _cheb_kernel,
        grid=(_NBLK + _NOUT,),
        in_specs=[
            pl.BlockSpec(
                (_BM, _N),
                lambda m: (jnp.minimum(m, _NBLK - 1), 0)),
            pl.BlockSpec((_N, _F), lambda m: (0, 0)),
            pl.BlockSpec((3, _F, _F), lambda m: (0, 0, 0)),
            pl.BlockSpec((1, _F), lambda m: (0, 0)),
        ],
        out_specs=pl.BlockSpec(
            (_BO, _F),
            lambda m: (jnp.maximum(m - _NBLK, 0), 0)),
        out_shape=jax.ShapeDtypeStruct((_N, _F), jnp.float32),
        scratch_shapes=[
            pltpu.VMEM((_N, _N), jnp.bfloat16),
            pltpu.VMEM((_N, _F), jnp.bfloat16),
        ],
        compiler_params=pltpu.CompilerParams(
            dimension_semantics=("arbitrary",),
        ),
    )(L_tilde, xb, weight, bias2d)
    return out


# speedup vs baseline: 1.1333x; 1.0079x over previous
import jax
import jax.numpy as jnp
from jax.experimental import pallas as pl
from jax.experimental.pallas import tpu as pltpu

_N = 4096
_F = 256
_BM = 512
_BO = 1024
_NBLK = _N // _BM
_NOUT = _N // _BO


def _cheb_kernel(L_ref, xb_ref, w_ref, b_ref, out_ref, Lb_ref, t1_ref, y_ref):
    m = pl.program_id(0)

    @pl.when(m < _NBLK)
    def _stream():
        row = pl.ds(m * _BM, _BM)
        strip = L_ref[...].astype(jnp.bfloat16)
        Lb_ref[row, :] = strip
        t1 = jnp.dot(strip, xb_ref[...], preferred_element_type=jnp.float32)
        t1_ref[row, :] = t1.astype(jnp.bfloat16)

    @pl.when(m == _NBLK)
    def _fold_w2():
        w2 = w_ref[2, :, :].astype(jnp.bfloat16)
        y = jnp.dot(t1_ref[...], w2, preferred_element_type=jnp.float32)
        y_ref[...] = (2.0 * y).astype(jnp.bfloat16)

    @pl.when(m >= _NBLK)
    def _produce():
        row = pl.ds((m - _NBLK) * _BO, _BO)
        w0m2 = (w_ref[0, :, :] - w_ref[2, :, :]).astype(jnp.bfloat16)
        w1 = w_ref[1, :, :].astype(jnp.bfloat16)
        acc = jnp.dot(Lb_ref[row, :], y_ref[...],
                      preferred_element_type=jnp.float32)
        acc += jnp.dot(xb_ref[row, :], w0m2, preferred_element_type=jnp.float32)
        acc += jnp.dot(t1_ref[row, :], w1, preferred_element_type=jnp.float32)
        out_ref[...] = acc + b_ref[...]


def kernel(x, L_tilde, weight, bias):
    xb = x.astype(jnp.bfloat16)
    bias2d = bias.reshape(1, _F)

    out = pl.pallas_call(
        _cheb_kernel,
        grid=(_NBLK + _NOUT,),
        in_specs=[
            pl.BlockSpec(
                (_BM, _N),
                lambda m: (jnp.minimum(m, _NBLK - 1), 0)),
            pl.BlockSpec((_N, _F), lambda m: (0, 0)),
            pl.BlockSpec((3, _F, _F), lambda m: (0, 0, 0)),
            pl.BlockSpec((1, _F), lambda m: (0, 0)),
        ],
        out_specs=pl.BlockSpec(
            (_BO, _F),
            lambda m: (jnp.maximum(m - _NBLK, 0), 0)),
        out_shape=jax.ShapeDtypeStruct((_N, _F), jnp.float32),
        scratch_shapes=[
            pltpu.VMEM((_N, _N), jnp.bfloat16),
            pltpu.VMEM((_N, _F), jnp.bfloat16),
            pltpu.VMEM((_N, _F), jnp.bfloat16),
        ],
        compiler_params=pltpu.CompilerParams(
            dimension_semantics=("arbitrary",),
        ),
    )(L_tilde, xb, weight, bias2d)
    return out
